# baseline (speedup 1.0000x reference)
import jax
import jax.numpy as jnp
from jax.experimental import pallas as pl
from jax.experimental.pallas import tpu as pltpu

_B = 16
_MAX_SEQ = 2048
_H = 8
_D = 128
_PAGE = 256
_T = 32
_PAGES_PER_SEQ = _MAX_SEQ // _PAGE
_NUM_PAGES = _B * _PAGES_PER_SEQ
_ROWS = _NUM_PAGES * _PAGE

_CROWS = 1024
_NCHUNK = _ROWS // _CROWS
_NBUF = 12


def _body(tp_ref, s0_ref, k_hbm, v_hbm, kc_hbm, vc_hbm, ko_hbm, vo_hbm,
          bufs, ktok, vtok, in_sems, out_sems, tok_sem):
    ktok_cp = pltpu.make_async_copy(k_hbm, ktok, tok_sem)
    vtok_cp = pltpu.make_async_copy(v_hbm, vtok, tok_sem)
    ktok_cp.start()
    vtok_cp.start()

    tasks = [(kc_hbm, ko_hbm, i) for i in range(_NCHUNK)]
    tasks += [(vc_hbm, vo_hbm, i) for i in range(_NCHUNK)]
    nt = len(tasks)

    def in_cp(t):
        src, _, i = tasks[t]
        s = t % _NBUF
        return pltpu.make_async_copy(
            src.at[pl.ds(i * _CROWS, _CROWS)], bufs.at[s], in_sems.at[s])

    def out_cp(t):
        _, dst, i = tasks[t]
        s = t % _NBUF
        return pltpu.make_async_copy(
            bufs.at[s], dst.at[pl.ds(i * _CROWS, _CROWS)], out_sems.at[s])

    for t in range(min(_NBUF, nt)):
        in_cp(t).start()
    for t in range(nt):
        in_cp(t).wait()
        out_cp(t).start()
        nxt = t + _NBUF
        if nxt < nt:
            out_cp(t).wait()
            in_cp(nxt).start()
    for t in range(max(nt - _NBUF, 0), nt):
        out_cp(t).wait()

    ktok_cp.wait()
    vtok_cp.wait()
    toks = []
    for b in range(_B):
        dst = pl.multiple_of(tp_ref[b] * _PAGE + s0_ref[b], 8)
        toks.append(pltpu.make_async_copy(
            ktok.at[pl.ds(b * _T, _T)], ko_hbm.at[pl.ds(dst, _T)], tok_sem))
        toks.append(pltpu.make_async_copy(
            vtok.at[pl.ds(b * _T, _T)], vo_hbm.at[pl.ds(dst, _T)], tok_sem))
    for c in toks:
        c.start()
    for c in toks:
        c.wait()


def kernel(k, v, k_cache, v_cache, page_table, cache_seqlens):
    k2 = k.reshape(_B * _T, _H * _D)
    v2 = v.reshape(_B * _T, _H * _D)
    kc2 = k_cache.reshape(_ROWS, _H * _D)
    vc2 = v_cache.reshape(_ROWS, _H * _D)

    pos0 = cache_seqlens
    pg = pos0 // _PAGE
    tp = jnp.take_along_axis(page_table, pg[:, None], axis=1)[:, 0]
    s0 = pos0 % _PAGE

    ko2, vo2 = pl.pallas_call(
        _body,
        grid=(),
        in_specs=[
            pl.BlockSpec(memory_space=pltpu.SMEM),
            pl.BlockSpec(memory_space=pltpu.SMEM),
            pl.BlockSpec(memory_space=pl.ANY),
            pl.BlockSpec(memory_space=pl.ANY),
            pl.BlockSpec(memory_space=pl.ANY),
            pl.BlockSpec(memory_space=pl.ANY),
        ],
        out_specs=[
            pl.BlockSpec(memory_space=pl.ANY),
            pl.BlockSpec(memory_space=pl.ANY),
        ],
        out_shape=[
            jax.ShapeDtypeStruct((_ROWS, _H * _D), k_cache.dtype),
            jax.ShapeDtypeStruct((_ROWS, _H * _D), v_cache.dtype),
        ],
        scratch_shapes=[
            pltpu.VMEM((_NBUF, _CROWS, _H * _D), k_cache.dtype),
            pltpu.VMEM((_B * _T, _H * _D), k.dtype),
            pltpu.VMEM((_B * _T, _H * _D), v.dtype),
            pltpu.SemaphoreType.DMA((_NBUF,)),
            pltpu.SemaphoreType.DMA((_NBUF,)),
            pltpu.SemaphoreType.DMA,
        ],
    )(tp, s0, k2, v2, kc2, vc2)

    k_cache_new = ko2.reshape(_NUM_PAGES, _PAGE, _H, _D)
    v_cache_new = vo2.reshape(_NUM_PAGES, _PAGE, _H, _D)
    return (k_cache_new, v_cache_new, cache_seqlens + _T)


# speedup vs baseline: 1.4171x; 1.4171x over previous
import jax
import jax.numpy as jnp
from jax.experimental import pallas as pl
from jax.experimental.pallas import tpu as pltpu

_B = 16
_MAX_SEQ = 2048
_H = 8
_D = 128
_PAGE = 256
_T = 32
_PAGES_PER_SEQ = _MAX_SEQ // _PAGE
_NUM_PAGES = _B * _PAGES_PER_SEQ
_ROWS = _NUM_PAGES * _PAGE


def _body(tp_ref, s0_ref, k_hbm, v_hbm, kc_hbm, vc_hbm, ko_hbm, vo_hbm,
          ktok, vtok, tok_sem):
    del kc_hbm, vc_hbm
    ktok_cp = pltpu.make_async_copy(k_hbm, ktok, tok_sem)
    vtok_cp = pltpu.make_async_copy(v_hbm, vtok, tok_sem)
    ktok_cp.start()
    vtok_cp.start()
    ktok_cp.wait()
    vtok_cp.wait()
    toks = []
    for b in range(_B):
        dst = pl.multiple_of(tp_ref[b] * _PAGE + s0_ref[b], 8)
        toks.append(pltpu.make_async_copy(
            ktok.at[pl.ds(b * _T, _T)], ko_hbm.at[pl.ds(dst, _T)], tok_sem))
        toks.append(pltpu.make_async_copy(
            vtok.at[pl.ds(b * _T, _T)], vo_hbm.at[pl.ds(dst, _T)], tok_sem))
    for c in toks:
        c.start()
    for c in toks:
        c.wait()


def kernel(k, v, k_cache, v_cache, page_table, cache_seqlens):
    k2 = k.reshape(_B * _T, _H * _D)
    v2 = v.reshape(_B * _T, _H * _D)
    kc2 = k_cache.reshape(_ROWS, _H * _D)
    vc2 = v_cache.reshape(_ROWS, _H * _D)

    pos0 = cache_seqlens
    pg = pos0 // _PAGE
    tp = jnp.take_along_axis(page_table, pg[:, None], axis=1)[:, 0]
    s0 = pos0 % _PAGE

    ko2, vo2 = pl.pallas_call(
        _body,
        grid=(),
        in_specs=[
            pl.BlockSpec(memory_space=pltpu.SMEM),
            pl.BlockSpec(memory_space=pltpu.SMEM),
            pl.BlockSpec(memory_space=pl.ANY),
            pl.BlockSpec(memory_space=pl.ANY),
            pl.BlockSpec(memory_space=pl.ANY),
            pl.BlockSpec(memory_space=pl.ANY),
        ],
        out_specs=[
            pl.BlockSpec(memory_space=pl.ANY),
            pl.BlockSpec(memory_space=pl.ANY),
        ],
        out_shape=[
            jax.ShapeDtypeStruct((_ROWS, _H * _D), k_cache.dtype),
            jax.ShapeDtypeStruct((_ROWS, _H * _D), v_cache.dtype),
        ],
        input_output_aliases={4: 0, 5: 1},
        scratch_shapes=[
            pltpu.VMEM((_B * _T, _H * _D), k.dtype),
            pltpu.VMEM((_B * _T, _H * _D), v.dtype),
            pltpu.SemaphoreType.DMA,
        ],
    )(tp, s0, k2, v2, kc2, vc2)

    k_cache_new = ko2.reshape(_NUM_PAGES, _PAGE, _H, _D)
    v_cache_new = vo2.reshape(_NUM_PAGES, _PAGE, _H, _D)
    return (k_cache_new, v_cache_new, cache_seqlens + _T)


# speedup vs baseline: 1.4191x; 1.0014x over previous
import jax
import jax.numpy as jnp
from jax.experimental import pallas as pl
from jax.experimental.pallas import tpu as pltpu

_B = 16
_MAX_SEQ = 2048
_H = 8
_D = 128
_PAGE = 256
_T = 32
_PAGES_PER_SEQ = _MAX_SEQ // _PAGE
_NUM_PAGES = _B * _PAGES_PER_SEQ
_ROWS = _NUM_PAGES * _PAGE


def _body(tp_ref, s0_ref, k_hbm, v_hbm, kc_hbm, vc_hbm, ko_hbm, vo_hbm,
          ktok, vtok, tok_sem):
    del kc_hbm, vc_hbm
    ktok_cp = pltpu.make_async_copy(k_hbm, ktok, tok_sem)
    vtok_cp = pltpu.make_async_copy(v_hbm, vtok, tok_sem)
    ktok_cp.start()
    vtok_cp.start()
    ktok_cp.wait()
    vtok_cp.wait()
    toks = []
    for b in range(_B):
        dst = pl.multiple_of(tp_ref[b] * _PAGE + s0_ref[b], 8)
        toks.append(pltpu.make_async_copy(
            ktok.at[pl.ds(b * _T, _T)], ko_hbm.at[pl.ds(dst, _T)], tok_sem))
        toks.append(pltpu.make_async_copy(
            vtok.at[pl.ds(b * _T, _T)], vo_hbm.at[pl.ds(dst, _T)], tok_sem))
    for c in toks:
        c.start()
    for c in toks:
        c.wait()


def kernel(k, v, k_cache, v_cache, page_table, cache_seqlens):
    k2 = k.reshape(_B * _T, _H * _D)
    v2 = v.reshape(_B * _T, _H * _D)
    kc2 = jnp.copy(k_cache.reshape(_ROWS, _H * _D))
    vc2 = jnp.copy(v_cache.reshape(_ROWS, _H * _D))

    pos0 = cache_seqlens
    pg = pos0 // _PAGE
    tp = jnp.take_along_axis(page_table, pg[:, None], axis=1)[:, 0]
    s0 = pos0 % _PAGE

    ko2, vo2 = pl.pallas_call(
        _body,
        grid=(),
        in_specs=[
            pl.BlockSpec(memory_space=pltpu.SMEM),
            pl.BlockSpec(memory_space=pltpu.SMEM),
            pl.BlockSpec(memory_space=pl.ANY),
            pl.BlockSpec(memory_space=pl.ANY),
            pl.BlockSpec(memory_space=pl.ANY),
            pl.BlockSpec(memory_space=pl.ANY),
        ],
        out_specs=[
            pl.BlockSpec(memory_space=pl.ANY),
            pl.BlockSpec(memory_space=pl.ANY),
        ],
        out_shape=[
            jax.ShapeDtypeStruct((_ROWS, _H * _D), k_cache.dtype),
            jax.ShapeDtypeStruct((_ROWS, _H * _D), v_cache.dtype),
        ],
        input_output_aliases={4: 0, 5: 1},
        scratch_shapes=[
            pltpu.VMEM((_B * _T, _H * _D), k.dtype),
            pltpu.VMEM((_B * _T, _H * _D), v.dtype),
            pltpu.SemaphoreType.DMA,
        ],
    )(tp, s0, k2, v2, kc2, vc2)

    k_cache_new = ko2.reshape(_NUM_PAGES, _PAGE, _H, _D)
    v_cache_new = vo2.reshape(_NUM_PAGES, _PAGE, _H, _D)
    return (k_cache_new, v_cache_new, cache_seqlens + _T)
